# Initial kernel scaffold; baseline (speedup 1.0000x reference)
import jax
import jax.numpy as jnp
from jax.experimental import pallas as pl


def kernel(enc1_w, enc1_b, enc2_w, enc2_b, enc3_w, enc3_b, d1_w, d1_b, d2_w, d2_b, d3_w, d3_b, u3_w, u3_b, u2_w, u2_b, u1_w, u1_b, x):
    raise NotImplementedError("write your pallas kernel here")


# speedup vs baseline: 1.0002x; 1.0002x over previous
import reference as _r

def kernel(*args):
    return _r.reference(*args)


# speedup vs baseline: 2.7688x; 2.7683x over previous
import jax
import jax.numpy as jnp
from jax.experimental import pallas as pl
from jax.experimental.pallas import tpu as pltpu

_C, _CO = 64, 8
_H, _W = 256, 512
_OH, _OW = 128, 128
_T = 32
_NB = _OH // _T
_RPB = 2 * _T


def _enc1_kernel(xm_ref, xh_ref, w_ref, sel_ref, b_ref, o_ref, p_ref, u_ref):
    for r in range(_RPB):
        p_ref[r] = jnp.dot(w_ref[...], xm_ref[r],
                           preferred_element_type=jnp.float32)
    for r in range(2):
        p_ref[_RPB + r] = jnp.dot(w_ref[...], xh_ref[r],
                                  preferred_element_type=jnp.float32)
    for i in range(_T):
        u_ref[i] = (p_ref[2 * i, 0:32] + p_ref[2 * i + 1, 32:64]
                    + p_ref[2 * i + 2, 64:96] + p_ref[2 * i + 3, 96:128])
    acc = jnp.zeros((_T * _CO, _OW), jnp.float32)
    for dj in range(4):
        a = u_ref[:, 8 * dj:8 * dj + 8, :].reshape(_T * _CO, _W)
        acc = acc + jnp.dot(a, sel_ref[dj], preferred_element_type=jnp.float32)
    acc = acc + b_ref[...]
    acc = jnp.where(acc > 0, acc, 0.2 * acc)
    o_ref[...] = acc.reshape(_T, _CO, _OW).astype(o_ref.dtype)


def _enc1(x, w, bvec):
    B = x.shape[0]
    xt = jnp.transpose(x, (0, 2, 1, 3)).astype(jnp.bfloat16)
    xt = jnp.pad(xt, ((0, 0), (1, 7), (0, 0), (0, 0)))
    wall = jnp.transpose(w, (2, 3, 0, 1)).reshape(16 * _CO, _C)
    wall = wall.astype(jnp.bfloat16)
    m = jnp.arange(_W)[:, None]
    j = jnp.arange(_OW)[None, :]
    sel = jnp.stack([(m == 4 * j + dj - 1).astype(jnp.float32)
                     for dj in range(4)])
    bt = jnp.tile(bvec.astype(jnp.float32), _T).reshape(_T * _CO, 1)

    out = pl.pallas_call(
        _enc1_kernel,
        out_shape=jax.ShapeDtypeStruct((B, _NB, _T, _CO, _OW), jnp.bfloat16),
        grid=(B, _NB),
        in_specs=[
            pl.BlockSpec((None, _RPB, _C, _W), lambda b, k: (b, k, 0, 0)),
            pl.BlockSpec((None, 2, _C, _W), lambda b, k: (b, _T * (k + 1), 0, 0)),
            pl.BlockSpec((16 * _CO, _C), lambda b, k: (0, 0)),
            pl.BlockSpec((4, _W, _OW), lambda b, k: (0, 0, 0)),
            pl.BlockSpec((_T * _CO, 1), lambda b, k: (0, 0)),
        ],
        out_specs=pl.BlockSpec((None, None, _T, _CO, _OW),
                               lambda b, k: (b, k, 0, 0, 0)),
        scratch_shapes=[pltpu.VMEM((_RPB + 2, 16 * _CO, _W), jnp.float32),
                        pltpu.VMEM((_T, 32, _W), jnp.float32)],
        compiler_params=pltpu.CompilerParams(
            dimension_semantics=("parallel", "parallel")),
    )(xt, xt, wall, sel, bt)
    return out.reshape(B, _OH, _CO, _OW)


def _down_body(x, w_ref, sel_ref, b_ref, act, H, C, CO, W):
    OH, OW = H // 2, W // 2
    p = [jnp.dot(w_ref[...], x[r], preferred_element_type=jnp.float32)
         for r in range(H)]
    rows = []
    for i in range(OH):
        t = None
        for di in range(4):
            r = 2 * i + di - 1
            if 0 <= r < H:
                s = p[r][di * 4 * CO:(di + 1) * 4 * CO]
                t = s if t is None else t + s
        rows.append(t)
    u = jnp.stack(rows)
    acc = jnp.zeros((OH * CO, OW), jnp.float32)
    for dj in range(4):
        a = u[:, dj * CO:(dj + 1) * CO, :].reshape(OH * CO, W)
        acc = acc + jnp.dot(a, sel_ref[dj], preferred_element_type=jnp.float32)
    acc = acc + b_ref[...]
    if act == "leaky":
        acc = jnp.where(acc > 0, acc, 0.2 * acc)
    else:
        acc = jnp.maximum(acc, 0.0)
    return acc.reshape(OH, CO, OW)


def _up_body(x, w_ref, sel_ref, b_ref, act, H, C, CO, W):
    p = [jnp.dot(w_ref[...], x[r], preferred_element_type=jnp.float32)
         for r in range(H)]
    rows = []
    for a0 in range(H):
        t = None
        for ay in range(3):
            r = a0 + ay - 1
            if 0 <= r < H:
                s = p[r][ay * 12 * CO:(ay + 1) * 12 * CO]
                t = s if t is None else t + s
        rows.append(t)
    u = jnp.stack(rows)
    acc = jnp.zeros((H * 4 * CO, W), jnp.float32)
    for ax in range(3):
        a = u[:, ax * 4 * CO:(ax + 1) * 4 * CO, :].reshape(H * 4 * CO, W)
        acc = acc + jnp.dot(a, sel_ref[ax], preferred_element_type=jnp.float32)
    acc = acc + b_ref[...]
    if act == "relu":
        acc = jnp.maximum(acc, 0.0)
    else:
        acc = jnp.tanh(acc)
    v = acc.reshape(H, 2, 2, CO, W)
    v = jnp.transpose(v, (0, 1, 3, 4, 2))
    return v.reshape(2 * H, CO, 2 * W)


def _down_wall(w):
    cout, cin = w.shape[0], w.shape[1]
    return jnp.transpose(w, (2, 3, 0, 1)).reshape(16 * cout, cin)


def _down_sel(W, dtype=jnp.float32):
    m = jnp.arange(W)[:, None]
    j = jnp.arange(W // 2)[None, :]
    return jnp.stack([(m == 2 * j + dj - 1).astype(dtype) for dj in range(4)])


def _up_wall(w):
    cin, co = w.shape[:2]
    wph = jnp.zeros((3, 3, 2, 2, co, cin), w.dtype)
    for py in range(2):
        for px in range(2):
            for ay in range(3):
                ky = py + 3 - 2 * ay
                if not 0 <= ky < 4:
                    continue
                for ax in range(3):
                    kx = px + 3 - 2 * ax
                    if not 0 <= kx < 4:
                        continue
                    wph = wph.at[ay, ax, py, px].set(w[:, :, ky, kx].T)
    return wph.reshape(36 * co, cin)


def _up_sel(W, dtype=jnp.float32):
    m = jnp.arange(W)[:, None]
    b = jnp.arange(W)[None, :]
    return jnp.stack([(m == b + ax - 1).astype(dtype) for ax in range(3)])


def _tail_kernel(x_ref, w2_ref, s2_ref, b2_ref, w3_ref, s3_ref, b3_ref,
                 wd1_ref, sd1_ref, bd1_ref, wd2_ref, sd2_ref, bd2_ref,
                 wd3_ref, sd3_ref, bd3_ref, wu3_ref, su3_ref, bu3_ref,
                 wu2_ref, su2_ref, bu2_ref, wu1_ref, su1_ref, bu1_ref,
                 o_ref):
    x = x_ref[...]
    x = _down_body(x, w2_ref, s2_ref, b2_ref, "leaky", 128, 8, 16, 128)
    x = _down_body(x.astype(jnp.bfloat16), w3_ref, s3_ref, b3_ref,
                   "leaky", 64, 16, 32, 64)
    d1 = _down_body(x.astype(jnp.bfloat16), wd1_ref, sd1_ref, bd1_ref,
                    "leaky", 32, 32, 32, 32)
    d2 = _down_body(d1.astype(jnp.bfloat16), wd2_ref, sd2_ref, bd2_ref,
                    "leaky", 16, 32, 64, 16)
    d3 = _down_body(d2.astype(jnp.bfloat16), wd3_ref, sd3_ref, bd3_ref,
                    "relu", 8, 64, 64, 8)
    u3 = _up_body(d3.astype(jnp.bfloat16), wu3_ref, su3_ref, bu3_ref,
                  "relu", 4, 64, 64, 4)
    u3 = jnp.concatenate([u3, d2], axis=1)
    u2 = _up_body(u3.astype(jnp.bfloat16), wu2_ref, su2_ref, bu2_ref,
                  "relu", 8, 128, 32, 8)
    u2 = jnp.concatenate([u2, d1], axis=1)
    u1 = _up_body(u2.astype(jnp.bfloat16), wu1_ref, su1_ref, bu1_ref,
                  "tanh", 16, 64, 1, 16)
    o_ref[...] = u1.reshape(32, 32)


def kernel(enc1_w, enc1_b, enc2_w, enc2_b, enc3_w, enc3_b,
           d1_w, d1_b, d2_w, d2_b, d3_w, d3_b,
           u3_w, u3_b, u2_w, u2_b, u1_w, u1_b, x):
    x = x.reshape(-1, 64, 256, 512)
    B = x.shape[0]
    x1 = _enc1(x, enc1_w, enc1_b)

    bf = jnp.bfloat16
    ops = []
    for w, b, W, co, oh in ((enc2_w, enc2_b, 128, 16, 64),
                            (enc3_w, enc3_b, 64, 32, 32),
                            (d1_w, d1_b, 32, 32, 16),
                            (d2_w, d2_b, 16, 64, 8),
                            (d3_w, d3_b, 8, 64, 4)):
        ops += [_down_wall(w).astype(bf), _down_sel(W),
                jnp.tile(b.astype(jnp.float32), oh).reshape(oh * co, 1)]
    for w, b, W, co in ((u3_w, u3_b, 4, 64),
                        (u2_w, u2_b, 8, 32),
                        (u1_w, u1_b, 16, 1)):
        ops += [_up_wall(w).astype(bf), _up_sel(W),
                jnp.tile(jnp.tile(b.astype(jnp.float32), 4), W)
                .reshape(W * 4 * co, 1)]

    in_specs = [pl.BlockSpec((None, _OH, _CO, _OW), lambda b: (b, 0, 0, 0))]
    for op in ops:
        nd = op.ndim
        in_specs.append(pl.BlockSpec(op.shape,
                                     (lambda b: (0, 0)) if nd == 2
                                     else (lambda b: (0, 0, 0))))

    out = pl.pallas_call(
        _tail_kernel,
        out_shape=jax.ShapeDtypeStruct((B, 32, 32), jnp.float32),
        grid=(B,),
        in_specs=in_specs,
        out_specs=pl.BlockSpec((None, 32, 32), lambda b: (b, 0, 0)),
        compiler_params=pltpu.CompilerParams(
            dimension_semantics=("parallel",)),
    )(x1, *ops)
    return out.reshape(B, 1, 32, 32).astype(jnp.float32)


# speedup vs baseline: 4.8882x; 1.7654x over previous
import jax
import jax.numpy as jnp
from jax.experimental import pallas as pl
from jax.experimental.pallas import tpu as pltpu

_C, _CO = 64, 8
_H, _W = 256, 512
_OH, _OW = 128, 128
_T = 32
_NB = _OH // _T
_RPB = 2 * _T


def _enc1_kernel(xm_ref, xt_ref, xb_ref, w_ref, sel_ref, b_ref, o_ref,
                 p_ref, u_ref):
    k = pl.program_id(1)
    xr = jnp.transpose(xm_ref[...].astype(jnp.bfloat16), (1, 0, 2))
    xtop = jnp.transpose(xt_ref[...].astype(jnp.bfloat16), (1, 0, 2))
    xbot = jnp.transpose(xb_ref[...].astype(jnp.bfloat16), (1, 0, 2))
    for r in range(_RPB):
        p_ref[r + 1] = jnp.dot(w_ref[...], xr[r],
                               preferred_element_type=jnp.float32)
    top = jnp.dot(w_ref[...], xtop[7], preferred_element_type=jnp.float32)
    p_ref[0] = jnp.where(k > 0, top, 0.0)
    bot = jnp.dot(w_ref[...], xbot[0], preferred_element_type=jnp.float32)
    p_ref[_RPB + 1] = jnp.where(k < _NB - 1, bot, 0.0)
    for i in range(_T):
        u_ref[i] = (p_ref[2 * i, 0:32] + p_ref[2 * i + 1, 32:64]
                    + p_ref[2 * i + 2, 64:96] + p_ref[2 * i + 3, 96:128])
    acc = jnp.zeros((_T * _CO, _OW), jnp.float32)
    for dj in range(4):
        a = u_ref[:, 8 * dj:8 * dj + 8, :].reshape(_T * _CO, _W)
        acc = acc + jnp.dot(a, sel_ref[dj], preferred_element_type=jnp.float32)
    acc = acc + b_ref[...]
    acc = jnp.where(acc > 0, acc, 0.2 * acc)
    o_ref[...] = acc.reshape(_T, _CO, _OW).astype(o_ref.dtype)


def _enc1(x, w, bvec):
    B = x.shape[0]
    wall = jnp.transpose(w, (2, 3, 0, 1)).reshape(16 * _CO, _C)
    wall = wall.astype(jnp.bfloat16)
    m = jnp.arange(_W)[:, None]
    j = jnp.arange(_OW)[None, :]
    sel = jnp.stack([(m == 4 * j + dj - 1).astype(jnp.float32)
                     for dj in range(4)])
    bt = jnp.tile(bvec.astype(jnp.float32), _T).reshape(_T * _CO, 1)

    out = pl.pallas_call(
        _enc1_kernel,
        out_shape=jax.ShapeDtypeStruct((B, _NB, _T, _CO, _OW), jnp.bfloat16),
        grid=(B, _NB),
        in_specs=[
            pl.BlockSpec((None, _C, _RPB, _W), lambda b, k: (b, 0, k, 0)),
            pl.BlockSpec((None, _C, 8, _W),
                         lambda b, k: (b, 0, jnp.maximum(8 * k - 1, 0), 0)),
            pl.BlockSpec((None, _C, 8, _W),
                         lambda b, k: (b, 0, jnp.minimum(8 * k + 8, 31), 0)),
            pl.BlockSpec((16 * _CO, _C), lambda b, k: (0, 0)),
            pl.BlockSpec((4, _W, _OW), lambda b, k: (0, 0, 0)),
            pl.BlockSpec((_T * _CO, 1), lambda b, k: (0, 0)),
        ],
        out_specs=pl.BlockSpec((None, None, _T, _CO, _OW),
                               lambda b, k: (b, k, 0, 0, 0)),
        scratch_shapes=[pltpu.VMEM((_RPB + 2, 16 * _CO, _W), jnp.float32),
                        pltpu.VMEM((_T, 32, _W), jnp.float32)],
        compiler_params=pltpu.CompilerParams(
            dimension_semantics=("parallel", "parallel")),
    )(x, x, x, wall, sel, bt)
    return out.reshape(B, _OH, _CO, _OW)


def _down_body(x, w_ref, sel_ref, b_ref, act, H, C, CO, W):
    OH, OW = H // 2, W // 2
    p = [jnp.dot(w_ref[...], x[r], preferred_element_type=jnp.float32)
         for r in range(H)]
    rows = []
    for i in range(OH):
        t = None
        for di in range(4):
            r = 2 * i + di - 1
            if 0 <= r < H:
                s = p[r][di * 4 * CO:(di + 1) * 4 * CO]
                t = s if t is None else t + s
        rows.append(t)
    u = jnp.stack(rows)
    acc = jnp.zeros((OH * CO, OW), jnp.float32)
    for dj in range(4):
        a = u[:, dj * CO:(dj + 1) * CO, :].reshape(OH * CO, W)
        acc = acc + jnp.dot(a, sel_ref[dj], preferred_element_type=jnp.float32)
    acc = acc + b_ref[...]
    if act == "leaky":
        acc = jnp.where(acc > 0, acc, 0.2 * acc)
    else:
        acc = jnp.maximum(acc, 0.0)
    return acc.reshape(OH, CO, OW)


def _up_body(x, w_ref, sel_ref, b_ref, act, H, C, CO, W):
    p = [jnp.dot(w_ref[...], x[r], preferred_element_type=jnp.float32)
         for r in range(H)]
    rows = []
    for a0 in range(H):
        t = None
        for ay in range(3):
            r = a0 + ay - 1
            if 0 <= r < H:
                s = p[r][ay * 12 * CO:(ay + 1) * 12 * CO]
                t = s if t is None else t + s
        rows.append(t)
    u = jnp.stack(rows)
    acc = jnp.zeros((H * 4 * CO, W), jnp.float32)
    for ax in range(3):
        a = u[:, ax * 4 * CO:(ax + 1) * 4 * CO, :].reshape(H * 4 * CO, W)
        acc = acc + jnp.dot(a, sel_ref[ax], preferred_element_type=jnp.float32)
    acc = acc + b_ref[...]
    if act == "relu":
        acc = jnp.maximum(acc, 0.0)
    else:
        acc = jnp.tanh(acc)
    v = acc.reshape(H, 2, 2, CO, W)
    v = jnp.transpose(v, (0, 1, 3, 4, 2))
    return v.reshape(2 * H, CO, 2 * W)


def _down_wall(w):
    cout, cin = w.shape[0], w.shape[1]
    return jnp.transpose(w, (2, 3, 0, 1)).reshape(16 * cout, cin)


def _down_sel(W, dtype=jnp.float32):
    m = jnp.arange(W)[:, None]
    j = jnp.arange(W // 2)[None, :]
    return jnp.stack([(m == 2 * j + dj - 1).astype(dtype) for dj in range(4)])


def _up_wall(w):
    cin, co = w.shape[:2]
    wph = jnp.zeros((3, 3, 2, 2, co, cin), w.dtype)
    for py in range(2):
        for px in range(2):
            for ay in range(3):
                ky = py + 3 - 2 * ay
                if not 0 <= ky < 4:
                    continue
                for ax in range(3):
                    kx = px + 3 - 2 * ax
                    if not 0 <= kx < 4:
                        continue
                    wph = wph.at[ay, ax, py, px].set(w[:, :, ky, kx].T)
    return wph.reshape(36 * co, cin)


def _up_sel(W, dtype=jnp.float32):
    m = jnp.arange(W)[:, None]
    b = jnp.arange(W)[None, :]
    return jnp.stack([(m == b + ax - 1).astype(dtype) for ax in range(3)])


def _tail_kernel(x_ref, w2_ref, s2_ref, b2_ref, w3_ref, s3_ref, b3_ref,
                 wd1_ref, sd1_ref, bd1_ref, wd2_ref, sd2_ref, bd2_ref,
                 wd3_ref, sd3_ref, bd3_ref, wu3_ref, su3_ref, bu3_ref,
                 wu2_ref, su2_ref, bu2_ref, wu1_ref, su1_ref, bu1_ref,
                 o_ref):
    x = x_ref[...]
    x = _down_body(x, w2_ref, s2_ref, b2_ref, "leaky", 128, 8, 16, 128)
    x = _down_body(x.astype(jnp.bfloat16), w3_ref, s3_ref, b3_ref,
                   "leaky", 64, 16, 32, 64)
    d1 = _down_body(x.astype(jnp.bfloat16), wd1_ref, sd1_ref, bd1_ref,
                    "leaky", 32, 32, 32, 32)
    d2 = _down_body(d1.astype(jnp.bfloat16), wd2_ref, sd2_ref, bd2_ref,
                    "leaky", 16, 32, 64, 16)
    d3 = _down_body(d2.astype(jnp.bfloat16), wd3_ref, sd3_ref, bd3_ref,
                    "relu", 8, 64, 64, 8)
    u3 = _up_body(d3.astype(jnp.bfloat16), wu3_ref, su3_ref, bu3_ref,
                  "relu", 4, 64, 64, 4)
    u3 = jnp.concatenate([u3, d2], axis=1)
    u2 = _up_body(u3.astype(jnp.bfloat16), wu2_ref, su2_ref, bu2_ref,
                  "relu", 8, 128, 32, 8)
    u2 = jnp.concatenate([u2, d1], axis=1)
    u1 = _up_body(u2.astype(jnp.bfloat16), wu1_ref, su1_ref, bu1_ref,
                  "tanh", 16, 64, 1, 16)
    o_ref[...] = u1.reshape(32, 32)


def kernel(enc1_w, enc1_b, enc2_w, enc2_b, enc3_w, enc3_b,
           d1_w, d1_b, d2_w, d2_b, d3_w, d3_b,
           u3_w, u3_b, u2_w, u2_b, u1_w, u1_b, x):
    x = x.reshape(-1, 64, 256, 512)
    B = x.shape[0]
    x1 = _enc1(x, enc1_w, enc1_b)

    bf = jnp.bfloat16
    ops = []
    for w, b, W, co, oh in ((enc2_w, enc2_b, 128, 16, 64),
                            (enc3_w, enc3_b, 64, 32, 32),
                            (d1_w, d1_b, 32, 32, 16),
                            (d2_w, d2_b, 16, 64, 8),
                            (d3_w, d3_b, 8, 64, 4)):
        ops += [_down_wall(w).astype(bf), _down_sel(W),
                jnp.tile(b.astype(jnp.float32), oh).reshape(oh * co, 1)]
    for w, b, W, co in ((u3_w, u3_b, 4, 64),
                        (u2_w, u2_b, 8, 32),
                        (u1_w, u1_b, 16, 1)):
        ops += [_up_wall(w).astype(bf), _up_sel(W),
                jnp.tile(jnp.tile(b.astype(jnp.float32), 4), W)
                .reshape(W * 4 * co, 1)]

    in_specs = [pl.BlockSpec((None, _OH, _CO, _OW), lambda b: (b, 0, 0, 0))]
    for op in ops:
        nd = op.ndim
        in_specs.append(pl.BlockSpec(op.shape,
                                     (lambda b: (0, 0)) if nd == 2
                                     else (lambda b: (0, 0, 0))))

    out = pl.pallas_call(
        _tail_kernel,
        out_shape=jax.ShapeDtypeStruct((B, 32, 32), jnp.float32),
        grid=(B,),
        in_specs=in_specs,
        out_specs=pl.BlockSpec((None, 32, 32), lambda b: (b, 0, 0)),
        compiler_params=pltpu.CompilerParams(
            dimension_semantics=("parallel",)),
    )(x1, *ops)
    return out.reshape(B, 1, 32, 32).astype(jnp.float32)
